# Initial kernel scaffold; baseline (speedup 1.0000x reference)
import jax
import jax.numpy as jnp
from jax.experimental import pallas as pl


def kernel(X_u, X_v, edge_index, W0, b0, W1, b1, W2, b2):
    raise NotImplementedError("write your pallas kernel here")


# speedup vs baseline: 4.6306x; 4.6306x over previous
import functools

import jax
import jax.numpy as jnp
from jax import lax
from jax.experimental import pallas as pl
from jax.experimental.pallas import tpu as pltpu
from jax.experimental.pallas import tpu_sc as plsc

N_U = 10000
N_V = 10000
E = 320000
D = 128

NC = 2
NS = 16
NW = NC * NS

EPW = E // NW
K = 80
CHUNKS = EPW // K
RPT = N_U // NS
ZR = 125


def _sc_scatter_stage(tmp, src_idx, dst_idx):
    mesh = plsc.VectorSubcoreMesh(core_axis_name="c", subcore_axis_name="s",
                                  num_cores=NC, num_subcores=NS)

    @functools.partial(
        pl.kernel,
        out_type=jax.ShapeDtypeStruct((NC, N_U, D), jnp.float32),
        mesh=mesh,
        scratch_types=[
            pltpu.VMEM((K,), jnp.int32),
            pltpu.VMEM((K,), jnp.int32),
            pltpu.VMEM((K, D), jnp.float32),
            pltpu.VMEM((ZR, D), jnp.float32),
            pltpu.VMEM_SHARED((N_U, D), jnp.float32),
            pltpu.SemaphoreType.DMA,
        ],
    )
    def stage(tmp_hbm, src_hbm, dst_hbm, out_hbm,
              sidx_v, didx_v, rows_v, zero_v, acc_sh, sem):
        c = lax.axis_index("c")
        s = lax.axis_index("s")
        wid = s * NC + c

        def zrow(i, _):
            def zcol(j, _):
                zero_v[i, pl.ds(j * 16, 16)] = jnp.zeros((16,), jnp.float32)
                return 0
            return lax.fori_loop(0, D // 16, zcol, 0)
        lax.fori_loop(0, ZR, zrow, 0)
        for z in range(RPT // ZR):
            pltpu.sync_copy(zero_v, acc_sh.at[pl.ds(s * RPT + z * ZR, ZR)])
        plsc.subcore_barrier()

        def body(j, _):
            base = wid * EPW + j * K
            pltpu.sync_copy(src_hbm.at[pl.ds(base, K)], sidx_v)
            pltpu.sync_copy(dst_hbm.at[pl.ds(base, K)], didx_v)
            pltpu.async_copy(tmp_hbm.at[sidx_v], rows_v, sem).wait()
            pltpu.sync_copy(rows_v, acc_sh.at[didx_v], add=True)
            return 0
        lax.fori_loop(0, CHUNKS, body, 0)
        plsc.subcore_barrier()

        @pl.when(s == 0)
        def _():
            pltpu.sync_copy(acc_sh, out_hbm.at[c])

    return stage(tmp, src_idx, dst_idx)


_BM = 2000


def _tc_mm_kernel(x_ref, w_ref, b_ref, o_ref):
    o_ref[...] = (jnp.dot(x_ref[...], w_ref[...],
                          preferred_element_type=jnp.float32)
                  + b_ref[...])


def _tc_mm(x, w, b):
    return pl.pallas_call(
        _tc_mm_kernel,
        out_shape=jax.ShapeDtypeStruct((x.shape[0], D), jnp.float32),
        grid=(x.shape[0] // _BM,),
        in_specs=[
            pl.BlockSpec((_BM, D), lambda i: (i, 0)),
            pl.BlockSpec((D, D), lambda i: (0, 0)),
            pl.BlockSpec((1, D), lambda i: (0, 0)),
        ],
        out_specs=pl.BlockSpec((_BM, D), lambda i: (i, 0)),
    )(x, w, b.reshape(1, D))


def _tc_mm_fused_kernel(p_ref, w_ref, b_ref, o_ref):
    s = p_ref[0] + p_ref[1]
    o_ref[...] = (jnp.dot(s, w_ref[...], preferred_element_type=jnp.float32)
                  + b_ref[...])


def _tc_mm_fused(p, w, b):
    return pl.pallas_call(
        _tc_mm_fused_kernel,
        out_shape=jax.ShapeDtypeStruct((p.shape[1], D), jnp.float32),
        grid=(p.shape[1] // _BM,),
        in_specs=[
            pl.BlockSpec((NC, _BM, D), lambda i: (0, i, 0)),
            pl.BlockSpec((D, D), lambda i: (0, 0)),
            pl.BlockSpec((1, D), lambda i: (0, 0)),
        ],
        out_specs=pl.BlockSpec((_BM, D), lambda i: (i, 0)),
    )(p, w, b.reshape(1, D))


def _tc_add_kernel(p_ref, o_ref):
    o_ref[...] = p_ref[0] + p_ref[1]


def _tc_add(p):
    return pl.pallas_call(
        _tc_add_kernel,
        out_shape=jax.ShapeDtypeStruct((p.shape[1], D), jnp.float32),
        grid=(p.shape[1] // _BM,),
        in_specs=[pl.BlockSpec((NC, _BM, D), lambda i: (0, i, 0))],
        out_specs=pl.BlockSpec((_BM, D), lambda i: (i, 0)),
    )(p)


def kernel(X_u, X_v, edge_index, W0, b0, W1, b1, W2, b2):
    u_idx = edge_index[0].astype(jnp.int32)
    v_idx = edge_index[1].astype(jnp.int32)

    tmp = _tc_mm(X_v, W0, b0)
    pu = _sc_scatter_stage(tmp, v_idx, u_idx)
    tmp = _tc_mm_fused(pu, W1, b1)
    pv = _sc_scatter_stage(tmp, u_idx, v_idx)
    tmp = _tc_mm_fused(pv, W2, b2)
    pu = _sc_scatter_stage(tmp, v_idx, u_idx)
    return _tc_add(pu)


# speedup vs baseline: 8.4732x; 1.8298x over previous
import functools

import jax
import jax.numpy as jnp
from jax import lax
from jax.experimental import pallas as pl
from jax.experimental.pallas import tpu as pltpu
from jax.experimental.pallas import tpu_sc as plsc

N_U = 10000
N_V = 10000
E = 320000
D = 128

NC = 2
NS = 16
NW = NC * NS

EPW = E // NW
K = 80
CHUNKS = EPW // K
BLK = 25
NBLK = CHUNKS // BLK
PAIRS = (BLK - 1) // 2
RPT = N_U // NS
ZR = 25


def _sc_scatter_stage(tmp, src_idx2d, dst_idx2d):
    mesh = plsc.VectorSubcoreMesh(core_axis_name="c", subcore_axis_name="s",
                                  num_cores=NC, num_subcores=NS)

    @functools.partial(
        pl.kernel,
        out_type=jax.ShapeDtypeStruct((NC, N_U, D), jnp.float32),
        mesh=mesh,
        scratch_types=[
            pltpu.VMEM((BLK, K), jnp.int32),
            pltpu.VMEM((BLK, K), jnp.int32),
            pltpu.VMEM((K, D), jnp.float32),
            pltpu.VMEM((K, D), jnp.float32),
            pltpu.VMEM((ZR, D), jnp.float32),
            pltpu.VMEM_SHARED((N_U, D), jnp.float32),
            pltpu.SemaphoreType.DMA,
            pltpu.SemaphoreType.DMA,
            pltpu.SemaphoreType.DMA,
        ],
    )
    def stage(tmp_hbm, src_hbm, dst_hbm, out_hbm,
              sidx_v, didx_v, rows_a, rows_b, zero_v, acc_sh,
              sem_a, sem_b, sem_i):
        c = lax.axis_index("c")
        s = lax.axis_index("s")
        wid = s * NC + c

        def load_idx(b):
            pltpu.async_copy(src_hbm.at[wid, b], sidx_v, sem_i)
            pltpu.async_copy(dst_hbm.at[wid, b], didx_v, sem_i)

        def wait_idx():
            pltpu.make_async_copy(src_hbm.at[0, 0], sidx_v, sem_i).wait()
            pltpu.make_async_copy(dst_hbm.at[0, 0], didx_v, sem_i).wait()

        load_idx(0)

        def zrow(i, _):
            def zcol(j, _):
                zero_v[i, pl.ds(j * 16, 16)] = jnp.zeros((16,), jnp.float32)
                return 0
            return lax.fori_loop(0, D // 16, zcol, 0)
        lax.fori_loop(0, ZR, zrow, 0)
        for z in range(RPT // ZR):
            pltpu.sync_copy(zero_v, acc_sh.at[pl.ds(s * RPT + z * ZR, ZR)])
        plsc.subcore_barrier()

        def gather(t, rows, sem):
            return pltpu.async_copy(tmp_hbm.at[sidx_v.at[t]], rows, sem)

        def wait_rows(rows, sem):
            pltpu.make_async_copy(tmp_hbm.at[sidx_v.at[0]], rows, sem).wait()

        def scatter(t, rows):
            pltpu.sync_copy(rows, acc_sh.at[didx_v.at[t]], add=True)

        for b in range(NBLK):
            wait_idx()
            gather(0, rows_a, sem_a)

            def body(m, _):
                t0 = 2 * m
                wait_rows(rows_a, sem_a)
                gather(t0 + 1, rows_b, sem_b)
                scatter(t0, rows_a)
                wait_rows(rows_b, sem_b)
                gather(t0 + 2, rows_a, sem_a)
                scatter(t0 + 1, rows_b)
                return 0
            lax.fori_loop(0, PAIRS, body, 0)
            wait_rows(rows_a, sem_a)
            scatter(BLK - 1, rows_a)
            if b + 1 < NBLK:
                load_idx(b + 1)
        plsc.subcore_barrier()

        @pl.when(s == 0)
        def _():
            pltpu.sync_copy(acc_sh, out_hbm.at[c])

    return stage(tmp, src_idx2d, dst_idx2d)


_BM = 2000


def _tc_mm_kernel(x_ref, w_ref, b_ref, o_ref):
    o_ref[...] = (jnp.dot(x_ref[...], w_ref[...],
                          preferred_element_type=jnp.float32)
                  + b_ref[...])


def _tc_mm(x, w, b):
    return pl.pallas_call(
        _tc_mm_kernel,
        out_shape=jax.ShapeDtypeStruct((x.shape[0], D), jnp.float32),
        grid=(x.shape[0] // _BM,),
        in_specs=[
            pl.BlockSpec((_BM, D), lambda i: (i, 0)),
            pl.BlockSpec((D, D), lambda i: (0, 0)),
            pl.BlockSpec((1, D), lambda i: (0, 0)),
        ],
        out_specs=pl.BlockSpec((_BM, D), lambda i: (i, 0)),
    )(x, w, b.reshape(1, D))


def _tc_mm_fused_kernel(p_ref, w_ref, b_ref, o_ref):
    s = p_ref[0] + p_ref[1]
    o_ref[...] = (jnp.dot(s, w_ref[...], preferred_element_type=jnp.float32)
                  + b_ref[...])


def _tc_mm_fused(p, w, b):
    return pl.pallas_call(
        _tc_mm_fused_kernel,
        out_shape=jax.ShapeDtypeStruct((p.shape[1], D), jnp.float32),
        grid=(p.shape[1] // _BM,),
        in_specs=[
            pl.BlockSpec((NC, _BM, D), lambda i: (0, i, 0)),
            pl.BlockSpec((D, D), lambda i: (0, 0)),
            pl.BlockSpec((1, D), lambda i: (0, 0)),
        ],
        out_specs=pl.BlockSpec((_BM, D), lambda i: (i, 0)),
    )(p, w, b.reshape(1, D))


def _tc_add_kernel(p_ref, o_ref):
    o_ref[...] = p_ref[0] + p_ref[1]


def _tc_add(p):
    return pl.pallas_call(
        _tc_add_kernel,
        out_shape=jax.ShapeDtypeStruct((p.shape[1], D), jnp.float32),
        grid=(p.shape[1] // _BM,),
        in_specs=[pl.BlockSpec((NC, _BM, D), lambda i: (0, i, 0))],
        out_specs=pl.BlockSpec((_BM, D), lambda i: (i, 0)),
    )(p)


def kernel(X_u, X_v, edge_index, W0, b0, W1, b1, W2, b2):
    u_idx = edge_index[0].astype(jnp.int32).reshape(NW, NBLK, BLK, K)
    v_idx = edge_index[1].astype(jnp.int32).reshape(NW, NBLK, BLK, K)

    tmp = _tc_mm(X_v, W0, b0)
    pu = _sc_scatter_stage(tmp, v_idx, u_idx)
    tmp = _tc_mm_fused(pu, W1, b1)
    pv = _sc_scatter_stage(tmp, u_idx, v_idx)
    tmp = _tc_mm_fused(pv, W2, b2)
    pu = _sc_scatter_stage(tmp, v_idx, u_idx)
    return _tc_add(pu)
